# baseline (speedup 1.0000x reference)
import functools

import jax
import jax.numpy as jnp
from jax import lax
from jax.experimental import pallas as pl
from jax.experimental.pallas import tpu as pltpu
from jax.experimental.pallas import tpu_sc as plsc

BATCH = 16384
EMB = 64
HID = 128

NC = 2
NS = 16
NW = NC * NS
NTAB = 1000000
SLAB = 128
NSLAB = 245
NB = 16
BSH = 11
MAXBASE = 1000064 - SLAB
CHUNK = 128
SENT = BATCH
OUTROWS = BATCH + 8


def _sc_gather(user, item, ut_t, it_t):
    mesh = plsc.VectorSubcoreMesh(core_axis_name="c", subcore_axis_name="s")

    @functools.partial(
        pl.kernel,
        mesh=mesh,
        compiler_params=pltpu.CompilerParams(needs_layout_passes=False),
        out_type=[
            jax.ShapeDtypeStruct((OUTROWS, 128), jnp.float32),
            jax.ShapeDtypeStruct((OUTROWS, 128), jnp.float32),
        ],
        scratch_types=[
            pltpu.VMEM((BATCH,), jnp.int32),
            pltpu.VMEM((BATCH,), jnp.int32),
            pltpu.VMEM((BATCH,), jnp.int32),
            pltpu.VMEM((BATCH,), jnp.int32),
            pltpu.VMEM((EMB, SLAB), jnp.float32),
            pltpu.VMEM((EMB, SLAB), jnp.float32),
            pltpu.VMEM((CHUNK, 128), jnp.float32),
            pltpu.VMEM((1, 128), jnp.int32),
            pltpu.VMEM((16,), jnp.int32),
            pltpu.VMEM((16,), jnp.int32),
            pltpu.VMEM((16,), jnp.int32),
            pltpu.SemaphoreType.DMA,
            pltpu.SemaphoreType.DMA,
            pltpu.SemaphoreType.DMA,
        ],
    )
    def gather_kernel(user_hbm, item_hbm, ut_hbm, it_hbm, uo_hbm, io_hbm,
                      idxall, match_r, match_j, mj2, bank_a, bank_b,
                      staging, jbuf, win_r, win_j, offs_v,
                      sem_a, sem_b, sem_o):
        wid = lax.axis_index("s") * NC + lax.axis_index("c")
        iota16 = lax.iota(jnp.int32, 16)
        start_w = wid * (NSLAB * SLAB)
        end_w = jnp.minimum(start_w + NSLAB * SLAB, NTAB)

        def slab_base(t):
            return pl.multiple_of(
                jnp.minimum(start_w + t * SLAB, MAXBASE), 128)

        def reset_jbuf():
            for mg in range(8):
                jbuf[0, pl.ds(16 * mg, 16)] = jnp.broadcast_to(SENT, (16,))

        def do_table(idx_hbm, tbl_hbm, out_hbm):
            pltpu.sync_copy(idx_hbm, idxall)
            reset_jbuf()

            def scan_body(i, cntm):
                v = idxall[pl.ds(i * 16, 16)]
                m = (v >= start_w) & (v < end_w)
                mi = m.astype(jnp.int32)
                pos = cntm + plsc.cumsum(mi) - mi
                plsc.store_scatter(match_r, [pos], v, mask=m)
                plsc.store_scatter(match_j, [pos], iota16 + 16 * i, mask=m)
                return cntm + plsc.all_reduce_population_count(m)[0]

            cntm = lax.fori_loop(0, BATCH // 16, scan_body, 0)
            nmv = (cntm + 15) >> 4

            off = 0
            for b in range(NB):
                def place_body(v, off, b=b):
                    rvec = match_r[pl.ds(v * 16, 16)]
                    valid = (iota16 + 16 * v) < cntm
                    bv = jnp.minimum((rvec - start_w) >> BSH, NB - 1)
                    m = valid & (bv == b)
                    mi = m.astype(jnp.int32)
                    pos = off + plsc.cumsum(mi) - mi
                    plsc.store_scatter(idxall, [pos], rvec, mask=m)
                    jm = match_j[pl.ds(v * 16, 16)]
                    plsc.store_scatter(mj2, [pos], jm, mask=m)
                    return off + plsc.all_reduce_population_count(m)[0]

                off = lax.fori_loop(0, nmv, place_body, off)
                plsc.store_scatter(
                    offs_v, [jnp.broadcast_to(b, (16,))],
                    jnp.broadcast_to(off, (16,)), mask=iota16 == 0)

            def flush():
                pltpu.async_copy(staging, out_hbm.at[jbuf.at[0]], sem_o).wait()
                reset_jbuf()

            def process(t, bank, cnt):
                base = slab_base(t)
                bid = jnp.minimum(t >> 4, NB - 1)
                end_b = plsc.load_gather(
                    offs_v, [jnp.broadcast_to(bid, (16,))])[0]
                prev = plsc.load_gather(
                    offs_v, [jnp.broadcast_to(jnp.maximum(bid - 1, 0), (16,))])[0]
                start_b = jnp.where(bid == 0, 0, prev)

                def win_body(v, cnt):
                    rvec = idxall[pl.ds(v * 16, 16)]
                    lanepos = iota16 + 16 * v
                    valid = (lanepos >= start_b) & (lanepos < end_b)
                    m = valid & (rvec >= base) & (rvec < base + SLAB)
                    mi = m.astype(jnp.int32)
                    wpos = plsc.cumsum(mi) - mi
                    plsc.store_scatter(win_r, [wpos], rvec, mask=m)
                    jm = mj2[pl.ds(v * 16, 16)]
                    plsc.store_scatter(win_j, [wpos], jm, mask=m)
                    hc = plsc.all_reduce_population_count(m)[0]

                    def hit_body(h, cnt):
                        hsp = jnp.broadcast_to(h, (16,))
                        colsp = plsc.load_gather(win_r, [hsp]) - base
                        jsp = plsc.load_gather(win_j, [hsp])
                        pos = cnt & (CHUNK - 1)
                        for mg in range(EMB // 16):
                            vm = plsc.load_gather(
                                bank, [iota16 + 16 * mg, colsp])
                            staging[pos, pl.ds(16 * mg, 16)] = vm
                        plsc.store_scatter(
                            jbuf.at[0], [jnp.broadcast_to(pos, (16,))],
                            jsp, mask=iota16 == 0)
                        cnt = cnt + 1

                        @pl.when((cnt & (CHUNK - 1)) == 0)
                        def _():
                            flush()

                        return cnt

                    return lax.fori_loop(0, hc, hit_body, cnt)

                return lax.fori_loop(start_b >> 4, (end_b + 15) >> 4,
                                     win_body, cnt)

            pltpu.async_copy(
                tbl_hbm.at[:, pl.ds(slab_base(0), SLAB)], bank_a, sem_a)
            pltpu.async_copy(
                tbl_hbm.at[:, pl.ds(slab_base(1), SLAB)], bank_b, sem_b)

            def slab_pair(u, cnt):
                t0 = 2 * u
                pltpu.make_async_copy(
                    tbl_hbm.at[:, pl.ds(0, SLAB)], bank_a, sem_a).wait()
                cnt = process(t0, bank_a, cnt)
                pltpu.async_copy(
                    tbl_hbm.at[:, pl.ds(slab_base(jnp.minimum(t0 + 2, NSLAB)),
                                        SLAB)], bank_a, sem_a)
                pltpu.make_async_copy(
                    tbl_hbm.at[:, pl.ds(0, SLAB)], bank_b, sem_b).wait()
                cnt = process(t0 + 1, bank_b, cnt)
                pltpu.async_copy(
                    tbl_hbm.at[:, pl.ds(slab_base(jnp.minimum(t0 + 3, NSLAB)),
                                        SLAB)], bank_b, sem_b)
                return cnt

            cnt = lax.fori_loop(0, (NSLAB + 1) // 2, slab_pair, 0)
            pltpu.make_async_copy(
                tbl_hbm.at[:, pl.ds(0, SLAB)], bank_a, sem_a).wait()
            pltpu.make_async_copy(
                tbl_hbm.at[:, pl.ds(0, SLAB)], bank_b, sem_b).wait()

            @pl.when((cnt & (CHUNK - 1)) != 0)
            def _():
                flush()

        do_table(user_hbm, ut_hbm, uo_hbm)
        do_table(item_hbm, it_hbm, io_hbm)

    return gather_kernel(user, item, ut_t, it_t)


BLK = 2048


def _mlp_body(u_ref, i_ref, w1u_ref, w1i_ref, b1_ref, w2_ref, b2_ref, o_ref):
    xu = lax.dot_general(u_ref[:, :EMB], w1u_ref[...], (((1,), (0,)), ((), ())),
                         preferred_element_type=jnp.float32)
    xi = lax.dot_general(i_ref[:, :EMB], w1i_ref[...], (((1,), (0,)), ((), ())),
                         preferred_element_type=jnp.float32)
    h = jnp.maximum(xu + xi + b1_ref[...], 0.0)
    y = lax.dot_general(h, w2_ref[...], (((1,), (0,)), ((), ())),
                        preferred_element_type=jnp.float32)
    o_ref[...] = 4.0 * jax.nn.sigmoid(y[:, 0:1] + b2_ref[0]) + 1.0


def _tc_mlp(uemb, iemb, w1u, w1i, b1, w2, b2):
    grid = (BATCH // BLK,)
    return pl.pallas_call(
        _mlp_body,
        grid=grid,
        in_specs=[
            pl.BlockSpec((BLK, 128), lambda b: (b, 0)),
            pl.BlockSpec((BLK, 128), lambda b: (b, 0)),
            pl.BlockSpec((EMB, HID), lambda b: (0, 0)),
            pl.BlockSpec((EMB, HID), lambda b: (0, 0)),
            pl.BlockSpec((1, HID), lambda b: (0, 0)),
            pl.BlockSpec((HID, 128), lambda b: (0, 0)),
            pl.BlockSpec(memory_space=pltpu.SMEM),
        ],
        out_specs=pl.BlockSpec((BLK, 1), lambda b: (b, 0)),
        out_shape=jax.ShapeDtypeStruct((BATCH, 1), jnp.float32),
    )(uemb, iemb, w1u, w1i, b1, w2, b2)


@jax.jit
def _run(user, item, user_table, item_table, W1, b1, W2, b2):
    uemb, iemb = _sc_gather(user.astype(jnp.int32), item.astype(jnp.int32),
                            user_table.T, item_table.T)
    w1u = W1[:, :EMB].T
    w1i = W1[:, EMB:].T
    w2pad = jnp.zeros((HID, 128), jnp.float32).at[:, 0].set(W2[0])
    out = _tc_mlp(uemb, iemb, w1u, w1i, b1.reshape(1, HID), w2pad, b2)
    return out.reshape(-1)


def kernel(user, item, user_table, item_table, W1, b1, W2, b2):
    return _run(user, item, user_table, item_table, W1, b1, W2, b2)


# speedup vs baseline: 1.2356x; 1.2356x over previous
import functools

import jax
import jax.numpy as jnp
from jax import lax
from jax.experimental import pallas as pl
from jax.experimental.pallas import tpu as pltpu
from jax.experimental.pallas import tpu_sc as plsc

BATCH = 16384
EMB = 64
HID = 128

NC = 2
NS = 16
NW = NC * NS
BPW = BATCH // NW
K = 4
NG = BPW // K
LG = 128


def _sc_gather(user, item, ut_t, it_t):
    mesh = plsc.VectorSubcoreMesh(core_axis_name="c", subcore_axis_name="s")

    @functools.partial(
        pl.kernel,
        mesh=mesh,
        compiler_params=pltpu.CompilerParams(needs_layout_passes=False),
        out_type=[
            jax.ShapeDtypeStruct((BATCH, EMB), jnp.float32),
            jax.ShapeDtypeStruct((BATCH, EMB), jnp.float32),
        ],
        scratch_types=[
            pltpu.VMEM((BPW,), jnp.int32),
            pltpu.VMEM((K, EMB, LG), jnp.float32),
            pltpu.VMEM((K, EMB, LG), jnp.float32),
            pltpu.VMEM((BPW // 2, EMB), jnp.float32),
            pltpu.SemaphoreType.DMA,
            pltpu.SemaphoreType.DMA,
        ],
    )
    def gather_kernel(user_hbm, item_hbm, ut_hbm, it_hbm, uo_hbm, io_hbm,
                      idx_v, bank_a, bank_b, rows_v, sem_a, sem_b):
        wid = lax.axis_index("s") * NC + lax.axis_index("c")
        base = wid * BPW
        iota16 = lax.iota(jnp.int32, 16)
        NS_SWEEPS = BPW // 16

        def do_table(idx_hbm, tbl_hbm, out_hbm):
            pltpu.sync_copy(idx_hbm.at[pl.ds(base, BPW)], idx_v)

            def fire(rbvec, q, bank, sem):
                for k in range(K):
                    rb = pl.multiple_of(rbvec[q * K + k], LG)
                    pltpu.async_copy(
                        tbl_hbm.at[:, pl.ds(rb, LG)], bank.at[k], sem)

            def drain(bank, sem):
                for k in range(K):
                    pltpu.make_async_copy(
                        tbl_hbm.at[:, pl.ds(0, LG)], bank.at[k], sem).wait()

            def extract(lvec, sweep, q, bank):
                for k in range(K):
                    j = (sweep % (NS_SWEEPS // 2)) * 16 + q * K + k
                    lane = jnp.broadcast_to(lvec[q * K + k], (16,))
                    for m in range(EMB // 16):
                        v = plsc.load_gather(
                            bank.at[k], [iota16 + 16 * m, lane])
                        rows_v[j, pl.ds(16 * m, 16)] = v

            def sweep_vecs(i):
                rvec = idx_v[pl.ds(i * 16, 16)]
                return (rvec >> 7) << 7, rvec & 127

            rb0, _ = sweep_vecs(0)
            fire(rb0, 0, bank_a, sem_a)
            fire(rb0, 1, bank_b, sem_b)

            def body(i, carry):
                @pl.when(i == NS_SWEEPS // 2)
                def _():
                    pltpu.sync_copy(rows_v, out_hbm.at[pl.ds(base, BPW // 2)])

                rbvec, lvec = sweep_vecs(i)
                inext = lax.rem(i + 1, NS_SWEEPS)
                rbnext, _ = sweep_vecs(inext)
                drain(bank_a, sem_a)
                extract(lvec, i, 0, bank_a)
                fire(rbvec, 2, bank_a, sem_a)
                drain(bank_b, sem_b)
                extract(lvec, i, 1, bank_b)
                fire(rbvec, 3, bank_b, sem_b)
                drain(bank_a, sem_a)
                extract(lvec, i, 2, bank_a)
                fire(rbnext, 0, bank_a, sem_a)
                drain(bank_b, sem_b)
                extract(lvec, i, 3, bank_b)
                fire(rbnext, 1, bank_b, sem_b)
                return carry

            lax.fori_loop(0, NS_SWEEPS, body, 0)
            drain(bank_a, sem_a)
            drain(bank_b, sem_b)
            pltpu.sync_copy(rows_v, out_hbm.at[pl.ds(base + BPW // 2, BPW // 2)])

        do_table(user_hbm, ut_hbm, uo_hbm)
        do_table(item_hbm, it_hbm, io_hbm)

    return gather_kernel(user, item, ut_t, it_t)


BLK = 2048


def _mlp_body(u_ref, i_ref, w1u_ref, w1i_ref, b1_ref, w2_ref, b2_ref, o_ref):
    xu = lax.dot_general(u_ref[...], w1u_ref[...], (((1,), (0,)), ((), ())),
                         preferred_element_type=jnp.float32)
    xi = lax.dot_general(i_ref[...], w1i_ref[...], (((1,), (0,)), ((), ())),
                         preferred_element_type=jnp.float32)
    h = jnp.maximum(xu + xi + b1_ref[...], 0.0)
    y = lax.dot_general(h, w2_ref[...], (((1,), (0,)), ((), ())),
                        preferred_element_type=jnp.float32)
    o_ref[...] = 4.0 * jax.nn.sigmoid(y[:, 0:1] + b2_ref[0]) + 1.0


def _tc_mlp(uemb, iemb, w1u, w1i, b1, w2, b2):
    grid = (BATCH // BLK,)
    return pl.pallas_call(
        _mlp_body,
        grid=grid,
        in_specs=[
            pl.BlockSpec((BLK, EMB), lambda b: (b, 0)),
            pl.BlockSpec((BLK, EMB), lambda b: (b, 0)),
            pl.BlockSpec((EMB, HID), lambda b: (0, 0)),
            pl.BlockSpec((EMB, HID), lambda b: (0, 0)),
            pl.BlockSpec((1, HID), lambda b: (0, 0)),
            pl.BlockSpec((HID, 128), lambda b: (0, 0)),
            pl.BlockSpec(memory_space=pltpu.SMEM),
        ],
        out_specs=pl.BlockSpec((BLK, 1), lambda b: (b, 0)),
        out_shape=jax.ShapeDtypeStruct((BATCH, 1), jnp.float32),
    )(uemb, iemb, w1u, w1i, b1, w2, b2)


@jax.jit
def _run(user, item, user_table, item_table, W1, b1, W2, b2):
    uemb, iemb = _sc_gather(user.astype(jnp.int32), item.astype(jnp.int32),
                            user_table.T, item_table.T)
    w1u = W1[:, :EMB].T
    w1i = W1[:, EMB:].T
    w2pad = jnp.zeros((HID, 128), jnp.float32).at[:, 0].set(W2[0])
    out = _tc_mlp(uemb, iemb, w1u, w1i, b1.reshape(1, HID), w2pad, b2)
    return out.reshape(-1)


def kernel(user, item, user_table, item_table, W1, b1, W2, b2):
    return _run(user, item, user_table, item_table, W1, b1, W2, b2)


# speedup vs baseline: 1.3609x; 1.1014x over previous
import functools

import jax
import jax.numpy as jnp
from jax import lax
from jax.experimental import pallas as pl
from jax.experimental.pallas import tpu as pltpu
from jax.experimental.pallas import tpu_sc as plsc

BATCH = 16384
EMB = 64
HID = 128

NC = 2
NS = 16
NW = NC * NS
BPW = BATCH // NW
K = 2
NG = BPW // K
LG = 128


def _sc_gather(user, item, ut_t, it_t):
    mesh = plsc.VectorSubcoreMesh(core_axis_name="c", subcore_axis_name="s")

    @functools.partial(
        pl.kernel,
        mesh=mesh,
        compiler_params=pltpu.CompilerParams(needs_layout_passes=False),
        out_type=[
            jax.ShapeDtypeStruct((BATCH, EMB), jnp.float32),
            jax.ShapeDtypeStruct((BATCH, EMB), jnp.float32),
        ],
        scratch_types=[
            pltpu.VMEM((BPW,), jnp.int32),
            pltpu.VMEM((K, EMB, LG), jnp.float32),
            pltpu.VMEM((K, EMB, LG), jnp.float32),
            pltpu.VMEM((K, EMB, LG), jnp.float32),
            pltpu.VMEM((K, EMB, LG), jnp.float32),
            pltpu.VMEM((BPW // 2, EMB), jnp.float32),
            pltpu.SemaphoreType.DMA,
            pltpu.SemaphoreType.DMA,
            pltpu.SemaphoreType.DMA,
            pltpu.SemaphoreType.DMA,
        ],
    )
    def gather_kernel(user_hbm, item_hbm, ut_hbm, it_hbm, uo_hbm, io_hbm,
                      idx_v, bank_a, bank_b, bank_c, bank_d, rows_v,
                      sem_a, sem_b, sem_c, sem_d):
        wid = lax.axis_index("s") * NC + lax.axis_index("c")
        base = wid * BPW
        iota16 = lax.iota(jnp.int32, 16)
        NS_SWEEPS = BPW // 16

        def do_table(idx_hbm, tbl_hbm, out_hbm):
            pltpu.sync_copy(idx_hbm.at[pl.ds(base, BPW)], idx_v)

            def fire(rbvec, q, bank, sem):
                for k in range(K):
                    rb = pl.multiple_of(rbvec[q * K + k], LG)
                    pltpu.async_copy(
                        tbl_hbm.at[:, pl.ds(rb, LG)], bank.at[k], sem)

            def drain(bank, sem):
                for k in range(K):
                    pltpu.make_async_copy(
                        tbl_hbm.at[:, pl.ds(0, LG)], bank.at[k], sem).wait()

            def extract(lvec, sweep, q, bank):
                for k in range(K):
                    j = (sweep % (NS_SWEEPS // 2)) * 16 + q * K + k
                    lane = jnp.broadcast_to(lvec[q * K + k], (16,))
                    for m in range(EMB // 16):
                        v = plsc.load_gather(
                            bank.at[k], [iota16 + 16 * m, lane])
                        rows_v[j, pl.ds(16 * m, 16)] = v

            def sweep_vecs(i):
                rvec = idx_v[pl.ds(i * 16, 16)]
                return (rvec >> 7) << 7, rvec & 127

            banks = [(bank_a, sem_a), (bank_b, sem_b),
                     (bank_c, sem_c), (bank_d, sem_d)]
            rb0, _ = sweep_vecs(0)
            for q in range(4):
                fire(rb0, q, *banks[q])

            def body(i, carry):
                @pl.when(i == NS_SWEEPS // 2)
                def _():
                    pltpu.sync_copy(rows_v, out_hbm.at[pl.ds(base, BPW // 2)])

                rbvec, lvec = sweep_vecs(i)
                inext = lax.rem(i + 1, NS_SWEEPS)
                rbnext, _ = sweep_vecs(inext)
                for q in range(4):
                    bank, sem = banks[q]
                    drain(bank, sem)
                    extract(lvec, i, q, bank)
                    fire(rbvec, 4 + q, bank, sem)
                for q in range(4):
                    bank, sem = banks[q]
                    drain(bank, sem)
                    extract(lvec, i, 4 + q, bank)
                    fire(rbnext, q, bank, sem)
                return carry

            lax.fori_loop(0, NS_SWEEPS, body, 0)
            for bank, sem in banks:
                drain(bank, sem)
            pltpu.sync_copy(rows_v, out_hbm.at[pl.ds(base + BPW // 2, BPW // 2)])

        do_table(user_hbm, ut_hbm, uo_hbm)
        do_table(item_hbm, it_hbm, io_hbm)

    return gather_kernel(user, item, ut_t, it_t)


BLK = 2048


def _mlp_body(u_ref, i_ref, w1u_ref, w1i_ref, b1_ref, w2_ref, b2_ref, o_ref):
    xu = lax.dot_general(u_ref[...], w1u_ref[...], (((1,), (0,)), ((), ())),
                         preferred_element_type=jnp.float32)
    xi = lax.dot_general(i_ref[...], w1i_ref[...], (((1,), (0,)), ((), ())),
                         preferred_element_type=jnp.float32)
    h = jnp.maximum(xu + xi + b1_ref[...], 0.0)
    y = lax.dot_general(h, w2_ref[...], (((1,), (0,)), ((), ())),
                        preferred_element_type=jnp.float32)
    o_ref[...] = 4.0 * jax.nn.sigmoid(y[:, 0:1] + b2_ref[0]) + 1.0


def _tc_mlp(uemb, iemb, w1u, w1i, b1, w2, b2):
    grid = (BATCH // BLK,)
    return pl.pallas_call(
        _mlp_body,
        grid=grid,
        in_specs=[
            pl.BlockSpec((BLK, EMB), lambda b: (b, 0)),
            pl.BlockSpec((BLK, EMB), lambda b: (b, 0)),
            pl.BlockSpec((EMB, HID), lambda b: (0, 0)),
            pl.BlockSpec((EMB, HID), lambda b: (0, 0)),
            pl.BlockSpec((1, HID), lambda b: (0, 0)),
            pl.BlockSpec((HID, 128), lambda b: (0, 0)),
            pl.BlockSpec(memory_space=pltpu.SMEM),
        ],
        out_specs=pl.BlockSpec((BLK, 1), lambda b: (b, 0)),
        out_shape=jax.ShapeDtypeStruct((BATCH, 1), jnp.float32),
    )(uemb, iemb, w1u, w1i, b1, w2, b2)


@jax.jit
def _run(user, item, user_table, item_table, W1, b1, W2, b2):
    uemb, iemb = _sc_gather(user.astype(jnp.int32), item.astype(jnp.int32),
                            user_table.T, item_table.T)
    w1u = W1[:, :EMB].T
    w1i = W1[:, EMB:].T
    w2pad = jnp.zeros((HID, 128), jnp.float32).at[:, 0].set(W2[0])
    out = _tc_mlp(uemb, iemb, w1u, w1i, b1.reshape(1, HID), w2pad, b2)
    return out.reshape(-1)


def kernel(user, item, user_table, item_table, W1, b1, W2, b2):
    return _run(user, item, user_table, item_table, W1, b1, W2, b2)


# speedup vs baseline: 1.4794x; 1.0871x over previous
import functools

import jax
import jax.numpy as jnp
from jax import lax
from jax.experimental import pallas as pl
from jax.experimental.pallas import tpu as pltpu
from jax.experimental.pallas import tpu_sc as plsc

BATCH = 16384
EMB = 64
HID = 128

NC = 2
NS = 16
NW = NC * NS
BPW = BATCH // NW
K = 1
NB8 = 8
LG = 128


def _sc_gather(user, item, ut_t, it_t):
    mesh = plsc.VectorSubcoreMesh(core_axis_name="c", subcore_axis_name="s")

    @functools.partial(
        pl.kernel,
        mesh=mesh,
        compiler_params=pltpu.CompilerParams(needs_layout_passes=False),
        out_type=[
            jax.ShapeDtypeStruct((BATCH, EMB), jnp.float32),
            jax.ShapeDtypeStruct((BATCH, EMB), jnp.float32),
        ],
        scratch_types=[
            pltpu.VMEM((BPW,), jnp.int32),
            pltpu.VMEM((NB8, EMB, LG), jnp.float32),
            pltpu.VMEM((BPW // 2, EMB), jnp.float32),
        ] + [pltpu.SemaphoreType.DMA] * NB8,
    )
    def gather_kernel(user_hbm, item_hbm, ut_hbm, it_hbm, uo_hbm, io_hbm,
                      idx_v, banks8, rows_v, *sems8):
        wid = lax.axis_index("s") * NC + lax.axis_index("c")
        base = wid * BPW
        iota16 = lax.iota(jnp.int32, 16)
        NS_SWEEPS = BPW // 16

        def do_table(idx_hbm, tbl_hbm, out_hbm):
            pltpu.sync_copy(idx_hbm.at[pl.ds(base, BPW)], idx_v)

            def fire(rbvec, q, b):
                rb = pl.multiple_of(rbvec[q], LG)
                pltpu.async_copy(
                    tbl_hbm.at[:, pl.ds(rb, LG)], banks8.at[b], sems8[b])

            def drain(b):
                pltpu.make_async_copy(
                    tbl_hbm.at[:, pl.ds(0, LG)], banks8.at[b],
                    sems8[b]).wait()

            def extract(lvec, sweep, q, b):
                j = (sweep % (NS_SWEEPS // 2)) * 16 + q
                lane = jnp.broadcast_to(lvec[q], (16,))
                for m in range(EMB // 16):
                    v = plsc.load_gather(
                        banks8.at[b], [iota16 + 16 * m, lane])
                    rows_v[j, pl.ds(16 * m, 16)] = v

            def sweep_vecs(i):
                rvec = idx_v[pl.ds(i * 16, 16)]
                return (rvec >> 7) << 7, rvec & 127

            rb0, _ = sweep_vecs(0)
            for q in range(8):
                fire(rb0, q, q)

            def body(i, carry):
                @pl.when(i == NS_SWEEPS // 2)
                def _():
                    pltpu.sync_copy(rows_v, out_hbm.at[pl.ds(base, BPW // 2)])

                rbvec, lvec = sweep_vecs(i)
                inext = lax.rem(i + 1, NS_SWEEPS)
                rbnext, _ = sweep_vecs(inext)
                for q in range(8):
                    drain(q)
                    extract(lvec, i, q, q)
                    fire(rbvec, 8 + q, q)
                for q in range(8):
                    drain(q)
                    extract(lvec, i, 8 + q, q)
                    fire(rbnext, q, q)
                return carry

            lax.fori_loop(0, NS_SWEEPS, body, 0)
            for q in range(8):
                drain(q)
            pltpu.sync_copy(rows_v, out_hbm.at[pl.ds(base + BPW // 2, BPW // 2)])

        do_table(user_hbm, ut_hbm, uo_hbm)
        do_table(item_hbm, it_hbm, io_hbm)

    return gather_kernel(user, item, ut_t, it_t)


BLK = 2048


def _mlp_body(u_ref, i_ref, w1u_ref, w1i_ref, b1_ref, w2_ref, b2_ref, o_ref):
    xu = lax.dot_general(u_ref[...], w1u_ref[...], (((1,), (0,)), ((), ())),
                         preferred_element_type=jnp.float32)
    xi = lax.dot_general(i_ref[...], w1i_ref[...], (((1,), (0,)), ((), ())),
                         preferred_element_type=jnp.float32)
    h = jnp.maximum(xu + xi + b1_ref[...], 0.0)
    y = lax.dot_general(h, w2_ref[...], (((1,), (0,)), ((), ())),
                        preferred_element_type=jnp.float32)
    o_ref[...] = 4.0 * jax.nn.sigmoid(y[:, 0:1] + b2_ref[0]) + 1.0


def _tc_mlp(uemb, iemb, w1u, w1i, b1, w2, b2):
    grid = (BATCH // BLK,)
    return pl.pallas_call(
        _mlp_body,
        grid=grid,
        in_specs=[
            pl.BlockSpec((BLK, EMB), lambda b: (b, 0)),
            pl.BlockSpec((BLK, EMB), lambda b: (b, 0)),
            pl.BlockSpec((EMB, HID), lambda b: (0, 0)),
            pl.BlockSpec((EMB, HID), lambda b: (0, 0)),
            pl.BlockSpec((1, HID), lambda b: (0, 0)),
            pl.BlockSpec((HID, 128), lambda b: (0, 0)),
            pl.BlockSpec(memory_space=pltpu.SMEM),
        ],
        out_specs=pl.BlockSpec((BLK, 1), lambda b: (b, 0)),
        out_shape=jax.ShapeDtypeStruct((BATCH, 1), jnp.float32),
    )(uemb, iemb, w1u, w1i, b1, w2, b2)


@jax.jit
def _run(user, item, user_table, item_table, W1, b1, W2, b2):
    uemb, iemb = _sc_gather(user.astype(jnp.int32), item.astype(jnp.int32),
                            user_table.T, item_table.T)
    w1u = W1[:, :EMB].T
    w1i = W1[:, EMB:].T
    w2pad = jnp.zeros((HID, 128), jnp.float32).at[:, 0].set(W2[0])
    out = _tc_mlp(uemb, iemb, w1u, w1i, b1.reshape(1, HID), w2pad, b2)
    return out.reshape(-1)


def kernel(user, item, user_table, item_table, W1, b1, W2, b2):
    return _run(user, item, user_table, item_table, W1, b1, W2, b2)


# speedup vs baseline: 1.4802x; 1.0005x over previous
import functools

import jax
import jax.numpy as jnp
from jax import lax
from jax.experimental import pallas as pl
from jax.experimental.pallas import tpu as pltpu
from jax.experimental.pallas import tpu_sc as plsc

BATCH = 16384
EMB = 64
HID = 128

NC = 2
NS = 16
NW = NC * NS
BPW = BATCH // NW
K = 1
NB8 = 8
LG = 128


def _sc_gather(user, item, ut_t, it_t):
    mesh = plsc.VectorSubcoreMesh(core_axis_name="c", subcore_axis_name="s")

    @functools.partial(
        pl.kernel,
        mesh=mesh,
        compiler_params=pltpu.CompilerParams(needs_layout_passes=False),
        out_type=[
            jax.ShapeDtypeStruct((BATCH, EMB), jnp.float32),
            jax.ShapeDtypeStruct((BATCH, EMB), jnp.float32),
        ],
        scratch_types=[
            pltpu.VMEM((BPW,), jnp.int32),
            pltpu.VMEM((NB8, EMB, LG), jnp.float32),
            pltpu.VMEM((BPW // 2, EMB), jnp.float32),
        ] + [pltpu.SemaphoreType.DMA] * NB8,
    )
    def gather_kernel(user_hbm, item_hbm, ut_hbm, it_hbm, uo_hbm, io_hbm,
                      idx_v, banks8, rows_v, *sems8):
        wid = lax.axis_index("s") * NC + lax.axis_index("c")
        base = wid * BPW
        iota16 = lax.iota(jnp.int32, 16)
        NS_SWEEPS = BPW // 16

        def do_table(idx_hbm, tbl_hbm, out_hbm):
            pltpu.sync_copy(idx_hbm.at[pl.ds(base, BPW)], idx_v)

            def fire(rbvec, q, b):
                rb = pl.multiple_of(rbvec[q], LG)
                for h in range(2):
                    pltpu.async_copy(
                        tbl_hbm.at[pl.ds(32 * h, 32), pl.ds(rb, LG)],
                        banks8.at[b].at[pl.ds(32 * h, 32)], sems8[b])

            def drain(b):
                pltpu.make_async_copy(
                    tbl_hbm.at[:, pl.ds(0, LG)], banks8.at[b],
                    sems8[b]).wait()

            def extract(lvec, sweep, q, b):
                j = (sweep % (NS_SWEEPS // 2)) * 16 + q
                lane = jnp.broadcast_to(lvec[q], (16,))
                for m in range(EMB // 16):
                    v = plsc.load_gather(
                        banks8.at[b], [iota16 + 16 * m, lane])
                    rows_v[j, pl.ds(16 * m, 16)] = v

            def sweep_vecs(i):
                rvec = idx_v[pl.ds(i * 16, 16)]
                return (rvec >> 7) << 7, rvec & 127

            rb0, _ = sweep_vecs(0)
            for q in range(8):
                fire(rb0, q, q)

            def body(i, carry):
                @pl.when(i == NS_SWEEPS // 2)
                def _():
                    pltpu.sync_copy(rows_v, out_hbm.at[pl.ds(base, BPW // 2)])

                rbvec, lvec = sweep_vecs(i)
                inext = lax.rem(i + 1, NS_SWEEPS)
                rbnext, _ = sweep_vecs(inext)
                for q in range(8):
                    drain(q)
                    extract(lvec, i, q, q)
                    fire(rbvec, 8 + q, q)
                for q in range(8):
                    drain(q)
                    extract(lvec, i, 8 + q, q)
                    fire(rbnext, q, q)
                return carry

            lax.fori_loop(0, NS_SWEEPS, body, 0)
            for q in range(8):
                drain(q)
            pltpu.sync_copy(rows_v, out_hbm.at[pl.ds(base + BPW // 2, BPW // 2)])

        do_table(user_hbm, ut_hbm, uo_hbm)
        do_table(item_hbm, it_hbm, io_hbm)

    return gather_kernel(user, item, ut_t, it_t)


BLK = 2048


def _mlp_body(u_ref, i_ref, w1u_ref, w1i_ref, b1_ref, w2_ref, b2_ref, o_ref):
    xu = lax.dot_general(u_ref[...], w1u_ref[...], (((1,), (0,)), ((), ())),
                         preferred_element_type=jnp.float32)
    xi = lax.dot_general(i_ref[...], w1i_ref[...], (((1,), (0,)), ((), ())),
                         preferred_element_type=jnp.float32)
    h = jnp.maximum(xu + xi + b1_ref[...], 0.0)
    y = lax.dot_general(h, w2_ref[...], (((1,), (0,)), ((), ())),
                        preferred_element_type=jnp.float32)
    o_ref[...] = 4.0 * jax.nn.sigmoid(y[:, 0:1] + b2_ref[0]) + 1.0


def _tc_mlp(uemb, iemb, w1u, w1i, b1, w2, b2):
    grid = (BATCH // BLK,)
    return pl.pallas_call(
        _mlp_body,
        grid=grid,
        in_specs=[
            pl.BlockSpec((BLK, EMB), lambda b: (b, 0)),
            pl.BlockSpec((BLK, EMB), lambda b: (b, 0)),
            pl.BlockSpec((EMB, HID), lambda b: (0, 0)),
            pl.BlockSpec((EMB, HID), lambda b: (0, 0)),
            pl.BlockSpec((1, HID), lambda b: (0, 0)),
            pl.BlockSpec((HID, 128), lambda b: (0, 0)),
            pl.BlockSpec(memory_space=pltpu.SMEM),
        ],
        out_specs=pl.BlockSpec((BLK, 1), lambda b: (b, 0)),
        out_shape=jax.ShapeDtypeStruct((BATCH, 1), jnp.float32),
    )(uemb, iemb, w1u, w1i, b1, w2, b2)


@jax.jit
def _run(user, item, user_table, item_table, W1, b1, W2, b2):
    uemb, iemb = _sc_gather(user.astype(jnp.int32), item.astype(jnp.int32),
                            user_table.T, item_table.T)
    w1u = W1[:, :EMB].T
    w1i = W1[:, EMB:].T
    w2pad = jnp.zeros((HID, 128), jnp.float32).at[:, 0].set(W2[0])
    out = _tc_mlp(uemb, iemb, w1u, w1i, b1.reshape(1, HID), w2pad, b2)
    return out.reshape(-1)


def kernel(user, item, user_table, item_table, W1, b1, W2, b2):
    return _run(user, item, user_table, item_table, W1, b1, W2, b2)
